# Initial kernel scaffold; baseline (speedup 1.0000x reference)
import functools

import jax
import jax.numpy as jnp
from jax import lax
from jax.experimental import pallas as pl
from jax.experimental.pallas import tpu as pltpu
from jax.experimental.pallas import tpu_sc as plsc

_NUM_CAMS = 16
_D = 2
_N = 16384 * 200
_NW = 32
_PER_W = _N // _NW
_C = 12800
_NCH = _PER_W // _C
_VI = _C // 16


def _sc_lookup():
    mesh = plsc.VectorSubcoreMesh(core_axis_name="c", subcore_axis_name="s")

    @functools.partial(
        pl.kernel,
        mesh=mesh,
        out_type=jax.ShapeDtypeStruct((_N * _D,), jnp.float32),
        scratch_types=[
            pltpu.VMEM((2 * _NUM_CAMS,), jnp.float32),
            pltpu.VMEM((_C,), jnp.int32),
            pltpu.VMEM((_C * _D,), jnp.float32),
        ],
    )
    def k(tbl_hbm, idx_hbm, out_hbm, tbl_v, idx_v, out_v):
        wid = lax.axis_index("s") * 2 + lax.axis_index("c")
        base = wid * _PER_W

        pltpu.sync_copy(tbl_hbm, tbl_v)
        evens = lax.iota(jnp.int32, 16) * 2
        sixteen = jnp.full((16,), _NUM_CAMS, jnp.int32)

        def chunk(g, _):
            pltpu.sync_copy(idx_hbm.at[pl.ds(base + g * _C, _C)], idx_v)

            def step(j, _):
                off = pl.multiple_of(j * 16, 16)
                iv = idx_v[pl.ds(off, 16)]
                v0 = plsc.load_gather(tbl_v, [iv])
                v1 = plsc.load_gather(tbl_v, [iv + sixteen])
                ob = evens + off * 2
                plsc.store_scatter(out_v, [ob], v0)
                plsc.store_scatter(out_v, [ob + 1], v1)
                return 0

            lax.fori_loop(0, _VI, step, 0)
            pltpu.sync_copy(out_v, out_hbm.at[pl.ds((base + g * _C) * _D, _C * _D)])
            return 0

        lax.fori_loop(0, _NCH, chunk, 0)

    return k


_lookup = _sc_lookup()


@jax.jit
def kernel(i, param):
    tbl = jnp.concatenate([param[:, 0], param[:, 1]]).astype(jnp.float32)
    iv = i.reshape(-1).astype(jnp.int32)
    out = _lookup(tbl, iv)
    return out.reshape(i.shape[0], i.shape[1], _D)


# speedup vs baseline: 5.4605x; 5.4605x over previous
import functools

import jax
import jax.numpy as jnp
from jax import lax
from jax.experimental import pallas as pl
from jax.experimental.pallas import tpu as pltpu
from jax.experimental.pallas import tpu_sc as plsc

_NUM_CAMS = 16
_D = 2
_N = 16384 * 200
_NW = 32
_PER_W = _N // _NW
_C = 12800
_NCH = _PER_W // _C
_VI = _C // 16


def _sc_lookup():
    mesh = plsc.VectorSubcoreMesh(core_axis_name="c", subcore_axis_name="s")

    @functools.partial(
        pl.kernel,
        mesh=mesh,
        out_type=jax.ShapeDtypeStruct((_N * _D,), jnp.float32),
        compiler_params=pltpu.CompilerParams(needs_layout_passes=False),
        scratch_types=[
            pltpu.VMEM((2 * _NUM_CAMS,), jnp.float32),
            pltpu.VMEM((_C,), jnp.int32),
            pltpu.VMEM((_C * _D,), jnp.float32),
        ],
    )
    def k(tbl_hbm, idx_hbm, out_hbm, tbl_v, idx_v, out_v):
        wid = lax.axis_index("s") * 2 + lax.axis_index("c")
        base = wid * _PER_W

        pltpu.sync_copy(tbl_hbm, tbl_v)
        evens = lax.iota(jnp.int32, 16) * 2
        sixteen = jnp.full((16,), _NUM_CAMS, jnp.int32)

        def chunk(g, _):
            pltpu.sync_copy(idx_hbm.at[pl.ds(base + g * _C, _C)], idx_v)

            def step(j, _):
                off = pl.multiple_of(j * 16, 16)
                iv = idx_v[pl.ds(off, 16)]
                v0 = plsc.load_gather(tbl_v, [iv])
                v1 = plsc.load_gather(tbl_v, [iv + sixteen])
                ob = evens + off * 2
                plsc.store_scatter(out_v, [ob], v0)
                plsc.store_scatter(out_v, [ob + 1], v1)
                return 0

            lax.fori_loop(0, _VI, step, 0)
            pltpu.sync_copy(out_v, out_hbm.at[pl.ds((base + g * _C) * _D, _C * _D)])
            return 0

        lax.fori_loop(0, _NCH, chunk, 0)

    return k


_lookup = _sc_lookup()


@jax.jit
def kernel(i, param):
    tbl = jnp.concatenate([param[:, 0], param[:, 1]]).astype(jnp.float32)
    iv = i.reshape(-1).astype(jnp.int32)
    out = _lookup(tbl, iv)
    return out.reshape(i.shape[0], i.shape[1], _D)


# speedup vs baseline: 5.5829x; 1.0224x over previous
import functools

import jax
import jax.numpy as jnp
from jax import lax
from jax.experimental import pallas as pl
from jax.experimental.pallas import tpu as pltpu
from jax.experimental.pallas import tpu_sc as plsc

_NUM_CAMS = 16
_D = 2
_N = 16384 * 200
_NW = 32
_PER_W = _N // _NW
_C = 12800
_NCH = _PER_W // _C
_VI = _C // 16


def _sc_lookup():
    mesh = plsc.VectorSubcoreMesh(core_axis_name="c", subcore_axis_name="s")

    @functools.partial(
        pl.kernel,
        mesh=mesh,
        out_type=jax.ShapeDtypeStruct((_N * _D,), jnp.float32),
        compiler_params=pltpu.CompilerParams(needs_layout_passes=False),
        scratch_types=[
            pltpu.VMEM((2 * _NUM_CAMS,), jnp.float32),
            pltpu.VMEM((_C,), jnp.int32),
            pltpu.VMEM((_C * _D,), jnp.float32),
        ],
    )
    def k(tbl_hbm, idx_hbm, out_hbm, tbl_v, idx_v, out_v):
        wid = lax.axis_index("s") * 2 + lax.axis_index("c")
        base = wid * _PER_W

        pltpu.sync_copy(tbl_hbm, tbl_v)
        evens = lax.iota(jnp.int32, 16) * 2
        sixteen = jnp.full((16,), _NUM_CAMS, jnp.int32)

        def chunk(g, _):
            pltpu.sync_copy(idx_hbm.at[pl.ds(base + g * _C, _C)], idx_v)

            @plsc.parallel_loop(0, _C, 16, unroll=8)
            def step(off):
                iv = idx_v[pl.ds(off, 16)]
                v0 = plsc.load_gather(tbl_v, [iv])
                v1 = plsc.load_gather(tbl_v, [iv + sixteen])
                ob = evens + off * 2
                plsc.store_scatter(out_v, [ob], v0)
                plsc.store_scatter(out_v, [ob + 1], v1)

            pltpu.sync_copy(out_v, out_hbm.at[pl.ds((base + g * _C) * _D, _C * _D)])
            return 0

        lax.fori_loop(0, _NCH, chunk, 0)

    return k


_lookup = _sc_lookup()


@jax.jit
def kernel(i, param):
    tbl = jnp.concatenate([param[:, 0], param[:, 1]]).astype(jnp.float32)
    iv = i.reshape(-1).astype(jnp.int32)
    out = _lookup(tbl, iv)
    return out.reshape(i.shape[0], i.shape[1], _D)


# speedup vs baseline: 243.8553x; 43.6790x over previous
import functools

import jax
import jax.numpy as jnp
from jax import lax
from jax.experimental import pallas as pl
from jax.experimental.pallas import tpu as pltpu
from jax.experimental.pallas import tpu_sc as plsc

_NUM_CAMS = 16
_D = 2
_S = 200
_B = 16384
_NW = 32


def _sc_lookup():
    mesh = plsc.VectorSubcoreMesh(core_axis_name="c", subcore_axis_name="s")

    @functools.partial(
        pl.kernel,
        mesh=mesh,
        out_type=jax.ShapeDtypeStruct((_S, 2 * 128, 128), jnp.float32),
        compiler_params=pltpu.CompilerParams(needs_layout_passes=False),
        scratch_types=[
            pltpu.VMEM((2 * _NUM_CAMS,), jnp.float32),
            pltpu.VMEM((128, 128), jnp.int32),
            pltpu.VMEM((2 * 128, 128), jnp.float32),
        ],
    )
    def k(tbl_hbm, idx_hbm, out_hbm, tbl_v, idx_v, out_v):
        w = lax.axis_index("s") * 2 + lax.axis_index("c")
        pltpu.sync_copy(tbl_hbm, tbl_v)
        sixteen = jnp.full((16,), _NUM_CAMS, jnp.int32)

        start = w * 6 + jnp.minimum(w, 8)
        count = jnp.where(w < 8, 7, 6)

        def row(r, _):
            s = start + r
            st = s // 8
            ss = s % 8
            pltpu.sync_copy(idx_hbm.at[st, :, ss, :], idx_v)

            @plsc.parallel_loop(0, 128, 1, unroll=2)
            def u_loop(u):
                for kk in range(8):
                    iv = idx_v[u, pl.ds(kk * 16, 16)]
                    v0 = plsc.load_gather(tbl_v, [iv])
                    v1 = plsc.load_gather(tbl_v, [iv + sixteen])
                    out_v[2 * u, pl.ds(kk * 16, 16)] = v0
                    out_v[2 * u + 1, pl.ds(kk * 16, 16)] = v1

            pltpu.sync_copy(out_v, out_hbm.at[s])
            return 0

        lax.fori_loop(0, count, row, 0)

    return k


_lookup = _sc_lookup()


@jax.jit
def kernel(i, param):
    tbl = jnp.concatenate([param[:, 0], param[:, 1]]).astype(jnp.float32)
    idx4 = i.astype(jnp.int32).reshape(128, 128, 25, 8).transpose(2, 0, 3, 1)
    out3 = _lookup(tbl, idx4)
    out = out3.reshape(_S, 128, _D, 128).transpose(1, 3, 0, 2)
    return out.reshape(_B, _S, _D)


# speedup vs baseline: 267.8530x; 1.0984x over previous
import functools

import jax
import jax.numpy as jnp
from jax import lax
from jax.experimental import pallas as pl
from jax.experimental.pallas import tpu as pltpu
from jax.experimental.pallas import tpu_sc as plsc

_NUM_CAMS = 16
_D = 2
_S = 200
_B = 16384
_NW = 32
_UPW = 25


def _sc_lookup():
    mesh = plsc.VectorSubcoreMesh(core_axis_name="c", subcore_axis_name="s")

    @functools.partial(
        pl.kernel,
        mesh=mesh,
        out_type=jax.ShapeDtypeStruct((_S, 2 * 128, 128), jnp.float32),
        compiler_params=pltpu.CompilerParams(needs_layout_passes=False),
        scratch_types=[
            pltpu.VMEM((2 * _NUM_CAMS,), jnp.float32),
            pltpu.VMEM((2, 32, 128), jnp.int32),
            pltpu.VMEM((2, 64, 128), jnp.float32),
            pltpu.SemaphoreType.DMA,
            pltpu.SemaphoreType.DMA,
        ],
    )
    def k(tbl_hbm, idx_hbm, out_hbm, tbl_v, idx_v, out_v, sin, sout):
        w = lax.axis_index("s") * 2 + lax.axis_index("c")
        pltpu.sync_copy(tbl_hbm, tbl_v)
        sixteen = jnp.full((16,), _NUM_CAMS, jnp.int32)
        u0 = w * _UPW

        def in_slice(unit):
            s = unit // 4
            q = unit % 4
            return idx_hbm.at[s // 8, pl.ds(q * 32, 32), s % 8, :]

        def out_slice(unit):
            s = unit // 4
            q = unit % 4
            return out_hbm.at[s, pl.ds(q * 64, 64), :]

        pltpu.async_copy(in_slice(u0), idx_v.at[0], sin)

        def unit_step(g, _):
            buf = lax.rem(g, 2)
            pltpu.make_async_copy(in_slice(u0), idx_v.at[buf], sin).wait()

            @pl.when(g < _UPW - 1)
            def _():
                pltpu.async_copy(in_slice(u0 + g + 1), idx_v.at[1 - buf], sin)

            @pl.when(g >= 2)
            def _():
                pltpu.make_async_copy(out_v.at[buf], out_slice(u0), sout).wait()

            @plsc.parallel_loop(0, 32, 1, unroll=2)
            def u_loop(u):
                for kk in range(8):
                    iv = idx_v[buf, u, pl.ds(kk * 16, 16)]
                    v0 = plsc.load_gather(tbl_v, [iv])
                    v1 = plsc.load_gather(tbl_v, [iv + sixteen])
                    out_v[buf, 2 * u, pl.ds(kk * 16, 16)] = v0
                    out_v[buf, 2 * u + 1, pl.ds(kk * 16, 16)] = v1

            pltpu.async_copy(out_v.at[buf], out_slice(u0 + g), sout)
            return 0

        lax.fori_loop(0, _UPW, unit_step, 0)
        pltpu.make_async_copy(out_v.at[0], out_slice(u0), sout).wait()
        pltpu.make_async_copy(out_v.at[0], out_slice(u0), sout).wait()

    return k


_lookup = _sc_lookup()


@jax.jit
def kernel(i, param):
    tbl = jnp.concatenate([param[:, 0], param[:, 1]]).astype(jnp.float32)
    idx4 = i.astype(jnp.int32).reshape(128, 128, 25, 8).transpose(2, 0, 3, 1)
    out3 = _lookup(tbl, idx4)
    out = out3.reshape(_S, 128, _D, 128).transpose(1, 3, 0, 2)
    return out.reshape(_B, _S, _D)


# speedup vs baseline: 269.9018x; 1.0076x over previous
import functools

import jax
import jax.numpy as jnp
from jax import lax
from jax.experimental import pallas as pl
from jax.experimental.pallas import tpu as pltpu
from jax.experimental.pallas import tpu_sc as plsc

_NUM_CAMS = 16
_D = 2
_S = 200
_B = 16384
_NW = 32
_UPW = 25


def _sc_lookup():
    mesh = plsc.VectorSubcoreMesh(core_axis_name="c", subcore_axis_name="s")

    @functools.partial(
        pl.kernel,
        mesh=mesh,
        out_type=jax.ShapeDtypeStruct((_S, 2 * 128, 128), jnp.float32),
        compiler_params=pltpu.CompilerParams(needs_layout_passes=False),
        scratch_types=[
            pltpu.VMEM((2 * _NUM_CAMS,), jnp.float32),
            pltpu.VMEM((2, 32, 128), jnp.int32),
            pltpu.VMEM((2, 64, 128), jnp.float32),
            pltpu.SemaphoreType.DMA,
            pltpu.SemaphoreType.DMA,
        ],
    )
    def k(tbl_hbm, idx_hbm, out_hbm, tbl_v, idx_v, out_v, sin, sout):
        w = lax.axis_index("s") * 2 + lax.axis_index("c")
        pltpu.sync_copy(tbl_hbm, tbl_v)
        tbl0 = tbl_v[pl.ds(0, 16)]
        tbl1 = tbl_v[pl.ds(16, 16)]
        u0 = w * _UPW

        def in_slice(unit):
            s = unit // 4
            q = unit % 4
            return idx_hbm.at[s // 8, pl.ds(q * 32, 32), s % 8, :]

        def out_slice(unit):
            s = unit // 4
            q = unit % 4
            return out_hbm.at[s, pl.ds(q * 64, 64), :]

        pltpu.async_copy(in_slice(u0), idx_v.at[0], sin)

        def unit_step(g, _):
            buf = lax.rem(g, 2)
            pltpu.make_async_copy(in_slice(u0), idx_v.at[buf], sin).wait()

            @pl.when(g < _UPW - 1)
            def _():
                pltpu.async_copy(in_slice(u0 + g + 1), idx_v.at[1 - buf], sin)

            @pl.when(g >= 2)
            def _():
                pltpu.make_async_copy(out_v.at[buf], out_slice(u0), sout).wait()

            @plsc.parallel_loop(0, 32, 1, unroll=4)
            def u_loop(u):
                for kk in range(8):
                    iv = idx_v[buf, u, pl.ds(kk * 16, 16)]
                    v0 = jnp.take_along_axis(tbl0, iv, axis=0)
                    v1 = jnp.take_along_axis(tbl1, iv, axis=0)
                    out_v[buf, 2 * u, pl.ds(kk * 16, 16)] = v0
                    out_v[buf, 2 * u + 1, pl.ds(kk * 16, 16)] = v1

            pltpu.async_copy(out_v.at[buf], out_slice(u0 + g), sout)
            return 0

        lax.fori_loop(0, _UPW, unit_step, 0)
        pltpu.make_async_copy(out_v.at[0], out_slice(u0), sout).wait()
        pltpu.make_async_copy(out_v.at[0], out_slice(u0), sout).wait()

    return k


_lookup = _sc_lookup()


@jax.jit
def kernel(i, param):
    tbl = jnp.concatenate([param[:, 0], param[:, 1]]).astype(jnp.float32)
    idx4 = i.astype(jnp.int32).reshape(128, 128, 25, 8).transpose(2, 0, 3, 1)
    out3 = _lookup(tbl, idx4)
    out = out3.reshape(_S, 128, _D, 128).transpose(1, 3, 0, 2)
    return out.reshape(_B, _S, _D)
